# baseline (speedup 1.0000x reference)
import functools

import jax
import jax.numpy as jnp
from jax import lax
from jax.experimental import pallas as pl
from jax.experimental.pallas import tpu as pltpu
from jax.experimental.pallas import tpu_sc as plsc


def _gather_call(B, V, D, NC, NS):
    NW = NC * NS
    b_per_w = B // NW
    mesh = plsc.VectorSubcoreMesh(core_axis_name="c", subcore_axis_name="s")

    @functools.partial(
        pl.kernel,
        mesh=mesh,
        compiler_params=pltpu.CompilerParams(use_tc_tiling_on_sc=False),
        out_type=jax.ShapeDtypeStruct((B, D), jnp.float32),
        scratch_types=[
            pltpu.VMEM((b_per_w,), jnp.int32),
            pltpu.VMEM((b_per_w, D), jnp.float32),
            pltpu.SemaphoreType.DMA,
        ],
    )
    def gather_kernel(ids_hbm, table_hbm, out_hbm, idx_v, rows_v, sem):
        wid = lax.axis_index("s") * NC + lax.axis_index("c")
        base = wid * b_per_w
        pltpu.sync_copy(ids_hbm.at[pl.ds(base, b_per_w)], idx_v)
        pltpu.async_copy(table_hbm.at[idx_v], rows_v, sem).wait()
        pltpu.sync_copy(rows_v, out_hbm.at[pl.ds(base, b_per_w)])

    return gather_kernel


def kernel(ids, table):
    B = ids.shape[0]
    V, D = table.shape
    info = plsc.get_sparse_core_info()
    return _gather_call(B, V, D, info.num_cores, info.num_subcores)(ids, table)


# speedup vs baseline: 4.1089x; 4.1089x over previous
import functools

import jax
import jax.numpy as jnp
from jax import lax
from jax.experimental import pallas as pl
from jax.experimental.pallas import tpu as pltpu
from jax.experimental.pallas import tpu_sc as plsc

_V = 1000000
_D = 32
_B = 16384
_K = 16


def _gather_call(NC, NS):
    NW = NC * NS
    b_per_w = _B // NW
    n_chunks = b_per_w // _K
    mesh = plsc.VectorSubcoreMesh(core_axis_name="c", subcore_axis_name="s")

    @functools.partial(
        pl.kernel,
        mesh=mesh,
        compiler_params=pltpu.CompilerParams(
            use_tc_tiling_on_sc=True, needs_layout_passes=False
        ),
        out_type=jax.ShapeDtypeStruct((_D, _B), jnp.float32),
        scratch_types=[
            pltpu.VMEM((b_per_w + _K,), jnp.int32),
            pltpu.VMEM((_K, _D, 128), jnp.float32),
            pltpu.VMEM((_D, b_per_w), jnp.float32),
            pltpu.SemaphoreType.DMA((_K,)),
        ],
    )
    def gather_kernel(ids_hbm, tableT_hbm, outT_hbm,
                      idx_v, ring, dst_v, sems):
        wid = lax.axis_index("s") * NC + lax.axis_index("c")
        base = wid * b_per_w

        pltpu.sync_copy(ids_hbm.at[pl.ds(base, b_per_w)],
                        idx_v.at[pl.ds(0, b_per_w)])

        def fetch(i, slot):
            c128 = pl.multiple_of((i >> 7) * 128, 128)
            pltpu.make_async_copy(
                tableT_hbm.at[:, pl.ds(c128, 128)],
                ring.at[slot],
                sems.at[slot],
            ).start()

        first = idx_v[pl.ds(0, _K)]
        for lane in range(_K):
            fetch(first[lane], lane)

        d0 = lax.iota(jnp.int32, 16)
        d1 = d0 + 16

        def step(ch, carry):
            cur = idx_v[pl.ds(ch * _K, _K)]
            nxt = idx_v[pl.ds((ch + 1) * _K, _K)]
            jbase = ch * _K
            for lane in range(_K):
                slot = lane
                pltpu.make_async_copy(
                    tableT_hbm.at[:, pl.ds(0, 128)],
                    ring.at[slot],
                    sems.at[slot],
                ).wait()
                c = jnp.broadcast_to(cur[lane] & 127, (16,))
                jv = jnp.broadcast_to(jbase + lane, (16,))
                v0 = plsc.load_gather(ring.at[slot], [d0, c])
                v1 = plsc.load_gather(ring.at[slot], [d1, c])
                plsc.store_scatter(dst_v, [d0, jv], v0)
                plsc.store_scatter(dst_v, [d1, jv], v1)

                @pl.when(ch < n_chunks - 1)
                def _():
                    fetch(nxt[lane], slot)

            return carry

        lax.fori_loop(0, n_chunks, step, 0)
        pltpu.sync_copy(dst_v, outT_hbm.at[:, pl.ds(base, b_per_w)])

    return gather_kernel


def kernel(ids, table):
    assert table.shape == (_V, _D) and ids.shape == (_B,)
    info = plsc.get_sparse_core_info()
    outT = _gather_call(info.num_cores, info.num_subcores)(ids, table.T)
    return outT.T
